# Initial kernel scaffold; baseline (speedup 1.0000x reference)
import jax
import jax.numpy as jnp
from jax.experimental import pallas as pl


def kernel(x, emb, W_c_w, W_c_b, Wih_f, Whh_f, bih_f, bhh_f, Wih_b, Whh_b, bih_b, bhh_b, W_out, b_out):
    raise NotImplementedError("write your pallas kernel here")


# speedup vs baseline: 3.8802x; 3.8802x over previous
import functools

import jax
import jax.numpy as jnp
from jax import lax
from jax.experimental import pallas as pl
from jax.experimental.pallas import tpu as pltpu
from jax.experimental.pallas import tpu_sc as plsc

V = 100000
E = 128
D = 128
H = 100
HP = 128
LBL = 104
B = 64
L = 50
NSLOT = 9
NNODE = B * L
NROWS = NSLOT * NNODE


def _ceil_to(x, m):
    return (x + m - 1) // m * m


def _sc_gather(idx_pad, table):
    info = plsc.get_sparse_core_info()
    nc, ns = info.num_cores, info.num_subcores
    nw = nc * ns
    npad = idx_pad.shape[0]
    bpw = npad // nw
    chunk = 128
    nfull = bpw // chunk
    rem = bpw - nfull * chunk

    mesh = plsc.VectorSubcoreMesh(core_axis_name="c", subcore_axis_name="s")

    @functools.partial(
        pl.kernel,
        mesh=mesh,
        out_type=jax.ShapeDtypeStruct((npad, E), jnp.float32),
        scratch_types=[
            pltpu.VMEM((bpw,), jnp.int32),
            pltpu.VMEM((bpw, E), jnp.float32),
            pltpu.SemaphoreType.DMA,
        ],
    )
    def gather_kernel(idx_hbm, table_hbm, out_hbm, idx_v, rows_v, sem):
        wid = lax.axis_index("s") * nc + lax.axis_index("c")
        base = wid * bpw
        pltpu.sync_copy(idx_hbm.at[pl.ds(base, bpw)], idx_v)
        copies = []
        for c in range(nfull):
            copies.append(pltpu.async_copy(
                table_hbm.at[idx_v.at[pl.ds(c * chunk, chunk)]],
                rows_v.at[pl.ds(c * chunk, chunk)], sem))
        if rem:
            copies.append(pltpu.async_copy(
                table_hbm.at[idx_v.at[pl.ds(nfull * chunk, rem)]],
                rows_v.at[pl.ds(nfull * chunk, rem)], sem))
        for cp in copies:
            cp.wait()
        pltpu.sync_copy(rows_v, out_hbm.at[pl.ds(base, bpw)])

    return gather_kernel(idx_pad, table)


def _dot_t(x, w):
    return lax.dot_general(x, w, (((1,), (1,)), ((), ())),
                           preferred_element_type=jnp.float32)


def _tc_body(g_ref, wc_ref, bc_ref,
             wihf_ref, whhf_ref, bihf_ref, bhhf_ref,
             wihb_ref, whhb_ref, bihb_ref, bhhb_ref,
             wout_ref, bout_ref,
             feat_ref, out_ref,
             gif_ref, gib_ref):
    bc = bc_ref[...]
    total = None
    mx = None
    for s in range(NSLOT):
        g = g_ref[s * NNODE:(s + 1) * NNODE, :]
        enc = _dot_t(g, wc_ref[...]) + bc
        if s == 0:
            total = enc
        else:
            total = total + enc
            mx = enc if s == 1 else jnp.maximum(mx, enc)
    encodes = jnp.maximum(total, mx)

    gif = _dot_t(encodes, wihf_ref[...]) + bihf_ref[...]
    gib = _dot_t(encodes, wihb_ref[...]) + bihb_ref[...]
    gif_ref[...] = gif.reshape(L, B, 3 * HP)
    gib_ref[...] = gib.reshape(L, B, 3 * HP)

    whhf = whhf_ref[...]
    whhb = whhb_ref[...]
    bhhf = bhhf_ref[...]
    bhhb = bhhb_ref[...]

    def step(t, carry):
        hf, hb, mf, mb = carry
        gf = gif_ref[t]
        gb = gib_ref[L - 1 - t]
        ghf = _dot_t(hf, whhf) + bhhf
        ghb = _dot_t(hb, whhb) + bhhb
        rf = jax.nn.sigmoid(gf[:, :HP] + ghf[:, :HP])
        zf = jax.nn.sigmoid(gf[:, HP:2 * HP] + ghf[:, HP:2 * HP])
        nf = jnp.tanh(gf[:, 2 * HP:] + rf * ghf[:, 2 * HP:])
        hf = (1.0 - zf) * nf + zf * hf
        rb = jax.nn.sigmoid(gb[:, :HP] + ghb[:, :HP])
        zb = jax.nn.sigmoid(gb[:, HP:2 * HP] + ghb[:, HP:2 * HP])
        nb = jnp.tanh(gb[:, 2 * HP:] + rb * ghb[:, 2 * HP:])
        hb = (1.0 - zb) * nb + zb * hb
        return hf, hb, jnp.maximum(mf, hf), jnp.maximum(mb, hb)

    z = jnp.zeros((B, HP), jnp.float32)
    neg = jnp.full((B, HP), -2.0, jnp.float32)
    _, _, mf, mb = lax.fori_loop(0, L, step, (z, z, neg, neg))
    feats = jnp.concatenate([mf, mb], axis=1)
    feat_ref[...] = feats
    out_ref[...] = _dot_t(feats, wout_ref[...]) + bout_ref[...]


def _pad_gates(w, n, cols_to=None):
    w3 = w.reshape(3, n, w.shape[1])
    w3 = jnp.pad(w3, ((0, 0), (0, HP - n), (0, 0)))
    if cols_to is not None:
        w3 = jnp.pad(w3, ((0, 0), (0, 0), (0, cols_to - w3.shape[2])))
    return w3.reshape(3 * HP, -1)


def _pad_gate_bias(b):
    return jnp.pad(b.reshape(3, H), ((0, 0), (0, HP - H))).reshape(1, 3 * HP)


def kernel(x, emb, W_c_w, W_c_b, Wih_f, Whh_f, bih_f, bhh_f,
           Wih_b, Whh_b, bih_b, bhh_b, W_out, b_out):
    idx = jnp.transpose(x, (2, 1, 0)).reshape(-1).astype(jnp.int32)
    npad = _ceil_to(NROWS, 32 * 8)
    idx_pad = jnp.pad(idx, (0, npad - NROWS))

    gathered = _sc_gather(idx_pad, emb)

    wihf = _pad_gates(Wih_f, H)
    wihb = _pad_gates(Wih_b, H)
    whhf = _pad_gates(Whh_f, H, cols_to=HP)
    whhb = _pad_gates(Whh_b, H, cols_to=HP)
    bihf = _pad_gate_bias(bih_f)
    bihb = _pad_gate_bias(bih_b)
    bhhf = _pad_gate_bias(bhh_f)
    bhhb = _pad_gate_bias(bhh_b)
    wout = jnp.pad(W_out.reshape(LBL, 2, H),
                   ((0, 0), (0, 0), (0, HP - H))).reshape(LBL, 2 * HP)
    bout = b_out.reshape(1, LBL)
    bc = W_c_b.reshape(1, D)

    feats_pad, outputs = pl.pallas_call(
        _tc_body,
        out_shape=(
            jax.ShapeDtypeStruct((B, 2 * HP), jnp.float32),
            jax.ShapeDtypeStruct((B, LBL), jnp.float32),
        ),
        scratch_shapes=[
            pltpu.VMEM((L, B, 3 * HP), jnp.float32),
            pltpu.VMEM((L, B, 3 * HP), jnp.float32),
        ],
    )(gathered, W_c_w, bc,
      wihf, whhf, bihf, bhhf,
      wihb, whhb, bihb, bhhb,
      wout, bout)

    features = jnp.concatenate(
        [feats_pad[:, :H], feats_pad[:, HP:HP + H]], axis=1)
    return (features, outputs)


# speedup vs baseline: 4.0663x; 1.0480x over previous
import functools

import jax
import jax.numpy as jnp
from jax import lax
from jax.experimental import pallas as pl
from jax.experimental.pallas import tpu as pltpu
from jax.experimental.pallas import tpu_sc as plsc

V = 100000
E = 128
D = 128
H = 100
HP = 128
LBL = 104
B = 64
L = 50
NSLOT = 9
NNODE = B * L
NROWS = NSLOT * NNODE


def _ceil_to(x, m):
    return (x + m - 1) // m * m


def _sc_gather(idx_pad, table):
    info = plsc.get_sparse_core_info()
    nc, ns = info.num_cores, info.num_subcores
    nw = nc * ns
    npad = idx_pad.shape[0]
    bpw = npad // nw
    chunk = 128
    nfull = bpw // chunk
    rem = bpw - nfull * chunk

    nchunks = nfull + (1 if rem else 0)
    sizes = [chunk] * nfull + ([rem] if rem else [])

    mesh = plsc.VectorSubcoreMesh(core_axis_name="c", subcore_axis_name="s")

    @functools.partial(
        pl.kernel,
        mesh=mesh,
        out_type=jax.ShapeDtypeStruct((npad, E), jnp.float32),
        scratch_types=[
            pltpu.VMEM((bpw,), jnp.int32),
            pltpu.VMEM((bpw, E), jnp.float32),
        ] + [pltpu.SemaphoreType.DMA] * (nchunks + 1),
    )
    def gather_kernel(idx_hbm, table_hbm, out_hbm, idx_v, rows_v, *sems):
        gsems, st_sem = sems[:nchunks], sems[nchunks]
        wid = lax.axis_index("s") * nc + lax.axis_index("c")
        base = wid * bpw
        pltpu.sync_copy(idx_hbm.at[pl.ds(base, bpw)], idx_v)
        copies = []
        for c in range(nchunks):
            copies.append(pltpu.async_copy(
                table_hbm.at[idx_v.at[pl.ds(c * chunk, sizes[c])]],
                rows_v.at[pl.ds(c * chunk, sizes[c])], gsems[c]))
        stores = []
        for c in range(nchunks):
            copies[c].wait()
            stores.append(pltpu.async_copy(
                rows_v.at[pl.ds(c * chunk, sizes[c])],
                out_hbm.at[pl.ds(base + c * chunk, sizes[c])], st_sem))
        for st in stores:
            st.wait()

    return gather_kernel(idx_pad, table)


def _dot_t(x, w):
    return lax.dot_general(x, w, (((1,), (1,)), ((), ())),
                           preferred_element_type=jnp.float32)


def _tc_body(g_ref, wc_ref, bc_ref,
             wihf_ref, whhf_ref, bihf_ref, bhhf_ref,
             wihb_ref, whhb_ref, bihb_ref, bhhb_ref,
             wout_ref, bout_ref,
             feat_ref, out_ref,
             gif_ref, gib_ref):
    bc = bc_ref[...]
    total = None
    mx = None
    for s in range(NSLOT):
        g = g_ref[s * NNODE:(s + 1) * NNODE, :]
        enc = _dot_t(g, wc_ref[...]) + bc
        if s == 0:
            total = enc
        else:
            total = total + enc
            mx = enc if s == 1 else jnp.maximum(mx, enc)
    encodes = jnp.maximum(total, mx)

    gif = _dot_t(encodes, wihf_ref[...]) + bihf_ref[...]
    gib = _dot_t(encodes, wihb_ref[...]) + bihb_ref[...]
    gif_ref[...] = gif.reshape(L, B, 3 * HP)
    gib_ref[...] = gib.reshape(L, B, 3 * HP)

    whhf = whhf_ref[...]
    whhb = whhb_ref[...]
    bhhf = bhhf_ref[...]
    bhhb = bhhb_ref[...]

    def step(t, carry):
        hf, hb, mf, mb = carry
        gf = gif_ref[t]
        gb = gib_ref[L - 1 - t]
        ghf = _dot_t(hf, whhf) + bhhf
        ghb = _dot_t(hb, whhb) + bhhb
        rf = jax.nn.sigmoid(gf[:, :HP] + ghf[:, :HP])
        zf = jax.nn.sigmoid(gf[:, HP:2 * HP] + ghf[:, HP:2 * HP])
        nf = jnp.tanh(gf[:, 2 * HP:] + rf * ghf[:, 2 * HP:])
        hf = (1.0 - zf) * nf + zf * hf
        rb = jax.nn.sigmoid(gb[:, :HP] + ghb[:, :HP])
        zb = jax.nn.sigmoid(gb[:, HP:2 * HP] + ghb[:, HP:2 * HP])
        nb = jnp.tanh(gb[:, 2 * HP:] + rb * ghb[:, 2 * HP:])
        hb = (1.0 - zb) * nb + zb * hb
        return hf, hb, jnp.maximum(mf, hf), jnp.maximum(mb, hb)

    z = jnp.zeros((B, HP), jnp.float32)
    neg = jnp.full((B, HP), -2.0, jnp.float32)
    _, _, mf, mb = lax.fori_loop(0, L, step, (z, z, neg, neg))
    feats = jnp.concatenate([mf, mb], axis=1)
    feat_ref[...] = feats
    out_ref[...] = _dot_t(feats, wout_ref[...]) + bout_ref[...]


def _pad_gates(w, n, cols_to=None):
    w3 = w.reshape(3, n, w.shape[1])
    w3 = jnp.pad(w3, ((0, 0), (0, HP - n), (0, 0)))
    if cols_to is not None:
        w3 = jnp.pad(w3, ((0, 0), (0, 0), (0, cols_to - w3.shape[2])))
    return w3.reshape(3 * HP, -1)


def _pad_gate_bias(b):
    return jnp.pad(b.reshape(3, H), ((0, 0), (0, HP - H))).reshape(1, 3 * HP)


def kernel(x, emb, W_c_w, W_c_b, Wih_f, Whh_f, bih_f, bhh_f,
           Wih_b, Whh_b, bih_b, bhh_b, W_out, b_out):
    idx = jnp.transpose(x, (2, 1, 0)).reshape(-1).astype(jnp.int32)
    npad = _ceil_to(NROWS, 32 * 8)
    idx_pad = jnp.pad(idx, (0, npad - NROWS))

    gathered = _sc_gather(idx_pad, emb)

    wihf = _pad_gates(Wih_f, H)
    wihb = _pad_gates(Wih_b, H)
    whhf = _pad_gates(Whh_f, H, cols_to=HP)
    whhb = _pad_gates(Whh_b, H, cols_to=HP)
    bihf = _pad_gate_bias(bih_f)
    bihb = _pad_gate_bias(bih_b)
    bhhf = _pad_gate_bias(bhh_f)
    bhhb = _pad_gate_bias(bhh_b)
    wout = jnp.pad(W_out.reshape(LBL, 2, H),
                   ((0, 0), (0, 0), (0, HP - H))).reshape(LBL, 2 * HP)
    bout = b_out.reshape(1, LBL)
    bc = W_c_b.reshape(1, D)

    feats_pad, outputs = pl.pallas_call(
        _tc_body,
        out_shape=(
            jax.ShapeDtypeStruct((B, 2 * HP), jnp.float32),
            jax.ShapeDtypeStruct((B, LBL), jnp.float32),
        ),
        scratch_shapes=[
            pltpu.VMEM((L, B, 3 * HP), jnp.float32),
            pltpu.VMEM((L, B, 3 * HP), jnp.float32),
        ],
    )(gathered, W_c_w, bc,
      wihf, whhf, bihf, bhhf,
      wihb, whhb, bihb, bhhb,
      wout, bout)

    features = jnp.concatenate(
        [feats_pad[:, :H], feats_pad[:, HP:HP + H]], axis=1)
    return (features, outputs)
